# baseline (speedup 1.0000x reference)
import functools

import jax
import jax.numpy as jnp
from jax import lax
from jax.experimental import pallas as pl
from jax.experimental.pallas import tpu as pltpu
from jax.experimental.pallas import tpu_sc as plsc

N = 8192

_info = plsc.get_sparse_core_info()
_NC, _NS, _L = _info.num_cores, _info.num_subcores, _info.num_lanes
_NW = _NC * _NS
_RPW = N // _NW
_RC = 4

_mesh = plsc.VectorSubcoreMesh(core_axis_name="c", subcore_axis_name="s")


@functools.partial(
    pl.kernel,
    mesh=_mesh,
    out_type=jax.ShapeDtypeStruct((N, N), jnp.float32),
    scratch_types=[
        pltpu.VMEM((N,), jnp.float32),
        pltpu.VMEM((_RC, N), jnp.float32),
        pltpu.VMEM((_RC, N), jnp.float32),
        pltpu.SemaphoreType.DMA,
        pltpu.SemaphoreType.DMA,
    ],
)
def _bt_sc(abil_hbm, out_hbm, abil_v, buf0, buf1, sem0, sem1):
    wid = lax.axis_index("s") * _NC + lax.axis_index("c")
    pltpu.sync_copy(abil_hbm, abil_v)
    base = wid * _RPW
    bufs, sems = (buf0, buf1), (sem0, sem1)

    def group_body(g, _):
        row0 = base + g * _L
        rv = abil_v[pl.ds(row0, _L)]
        for q in range(_L // _RC):
            buf, sem = bufs[q % 2], sems[q % 2]

            def drain(buf=buf, sem=sem):
                pltpu.make_async_copy(
                    buf, out_hbm.at[pl.ds(0, _RC)], sem).wait()

            if q < 2:
                pl.when(g > 0)(drain)
            else:
                drain()
            bs = [jnp.full((_L,), rv[q * _RC + rr], jnp.float32)
                  for rr in range(_RC)]

            def jbody(j, _, bs=bs, buf=buf):
                v = abil_v[pl.ds(j * _L, _L)]
                for rr in range(_RC):
                    buf[rr, pl.ds(j * _L, _L)] = (
                        1.0 / (1.0 + jnp.exp(v - bs[rr])))
                return 0

            lax.fori_loop(0, N // _L, jbody, 0, unroll=4)
            pltpu.async_copy(buf, out_hbm.at[pl.ds(row0 + q * _RC, _RC)], sem)
        return 0

    lax.fori_loop(0, _RPW // _L, group_body, 0)
    for b in range(2):
        pltpu.make_async_copy(bufs[b], out_hbm.at[pl.ds(0, _RC)], sems[b]).wait()


def kernel(ability):
    return _bt_sc(ability)


# speedup vs baseline: 2.4847x; 2.4847x over previous
import functools

import jax
import jax.numpy as jnp
from jax import lax
from jax.experimental import pallas as pl
from jax.experimental.pallas import tpu as pltpu
from jax.experimental.pallas import tpu_sc as plsc

N = 8192
SC_ROWS = 1024
BR = 256

_info = plsc.get_sparse_core_info()
_NC, _NS, _L = _info.num_cores, _info.num_subcores, _info.num_lanes
_NW = _NC * _NS
_RPW = SC_ROWS // _NW
_RC = 8

_mesh = plsc.VectorSubcoreMesh(core_axis_name="c", subcore_axis_name="s")


@functools.partial(
    pl.kernel,
    mesh=_mesh,
    out_type=jax.ShapeDtypeStruct((SC_ROWS, N), jnp.float32),
    scratch_types=[
        pltpu.VMEM((N,), jnp.float32),
        pltpu.VMEM((_RC, N), jnp.float32),
    ],
)
def _bt_sc(abil_hbm, out_hbm, abil_v, buf_v):
    wid = lax.axis_index("s") * _NC + lax.axis_index("c")
    pltpu.sync_copy(abil_hbm, abil_v)
    base = wid * _RPW

    def group_body(g, _):
        row0 = base + g * _L
        rv = abil_v[pl.ds(row0, _L)]
        for half in range(_L // _RC):
            bs = [jnp.full((_L,), rv[half * _RC + rr], jnp.float32)
                  for rr in range(_RC)]

            def jbody(j, _, bs=bs):
                v = abil_v[pl.ds(j * _L, _L)]
                for rr in range(_RC):
                    buf_v[rr, pl.ds(j * _L, _L)] = (
                        1.0 / (1.0 + jnp.exp(v - bs[rr])))
                return 0

            lax.fori_loop(0, N // _L, jbody, 0, unroll=2)
            pltpu.sync_copy(buf_v, out_hbm.at[pl.ds(row0 + half * _RC, _RC)])
        return 0

    lax.fori_loop(0, _RPW // _L, group_body, 0)


def _bt_tc_block(a_rows_ref, a_cols_ref, out_ref):
    nd = a_cols_ref[...] - a_rows_ref[...]
    out_ref[...] = 1.0 / (1.0 + jnp.exp(nd))


def _bt_tc(ability):
    a_rows = ability.reshape(N, 1)
    a_cols = ability.reshape(1, N)
    return pl.pallas_call(
        _bt_tc_block,
        grid=((N - SC_ROWS) // BR,),
        in_specs=[
            pl.BlockSpec((BR, 1), lambda i: (i + SC_ROWS // BR, 0)),
            pl.BlockSpec((1, N), lambda i: (0, 0)),
        ],
        out_specs=pl.BlockSpec((BR, N), lambda i: (i, 0)),
        out_shape=jax.ShapeDtypeStruct((N - SC_ROWS, N), jnp.float32),
    )(a_rows, a_cols)


def kernel(ability):
    sc_part = _bt_sc(ability)
    tc_part = _bt_tc(ability)
    return jnp.concatenate([sc_part, tc_part], axis=0)


# speedup vs baseline: 7.3863x; 2.9727x over previous
import jax
import jax.numpy as jnp
from jax.experimental import pallas as pl

N = 8192
BR = 512


def _bt_block(a_rows_ref, a_cols_ref, out_ref):
    nd = a_cols_ref[...] - a_rows_ref[...]
    out_ref[...] = 1.0 / (1.0 + jnp.exp(nd))


def kernel(ability):
    a_rows = ability.reshape(N, 1)
    a_cols = ability.reshape(1, N)
    return pl.pallas_call(
        _bt_block,
        grid=(N // BR,),
        in_specs=[
            pl.BlockSpec((BR, 1), lambda i: (i, 0)),
            pl.BlockSpec((1, N), lambda i: (0, 0)),
        ],
        out_specs=pl.BlockSpec((BR, N), lambda i: (i, 0)),
        out_shape=jax.ShapeDtypeStruct((N, N), jnp.float32),
    )(a_rows, a_cols)
